# baseline (speedup 1.0000x reference)
import functools
import math

import jax
import jax.numpy as jnp
from jax import lax
from jax.experimental import pallas as pl
from jax.experimental.pallas import tpu as pltpu
from jax.experimental.pallas import tpu_sc as plsc

_SIZE = 8192
_PAD = 0
_SMOOTHING = 0.1
_CONF = 1.0 - _SMOOTHING
_EPS = _SMOOTHING / (_SIZE - 2)
_C = (_SIZE - 2) * _EPS * math.log(_EPS) + _CONF * math.log(_CONF)

_ROWS_PER_BLOCK = 512

_NC, _NS, _L = 2, 16, 16
_NW = _NC * _NS
_SC_ROWS = 2048
_W_ROWS = _SC_ROWS // _NW


def _tc_dense_block(x_ref, t_ref, out_ref):
    i = pl.program_id(0)

    @pl.when(i == 0)
    def _():
        out_ref[0, 0] = 0.0

    xb = x_ref[...]
    t = t_ref[:, 0:1]
    rowsum = jnp.sum(xb, axis=1, keepdims=True)
    x0 = xb[:, 0:1]
    rb, v = xb.shape
    vocab_ids = lax.broadcasted_iota(jnp.int32, (rb, v), 1)
    xt = jnp.sum(jnp.where(vocab_ids == t, xb, 0.0), axis=1, keepdims=True)
    contrib = jnp.where(
        t >= 0,
        _C - _EPS * rowsum + _EPS * x0 - (_CONF - _EPS) * xt,
        0.0,
    )
    out_ref[0, 0] += jnp.sum(contrib)


_QW = _SIZE // 4
_NBUF = 4


def _sc_body(x_ref, te_ref, out_ref, tb_v, b0, b1, b2, b3, res_v, s0, s1, s2, s3):
    wid = lax.axis_index("s") * _NC + lax.axis_index("c")
    rows = x_ref.shape[0]
    r0 = rows - _SC_ROWS + wid * _W_ROWS
    nblocks = _W_ROWS // 8
    pltpu.sync_copy(
        te_ref.at[pl.ds(wid * _W_ROWS * _L, _W_ROWS * _L)], tb_v
    )
    lanes = lax.iota(jnp.int32, _L)
    lanes_c = [lanes + c * _L for c in range(8)]
    bufs = [b0, b1, b2, b3]
    sems = [s0, s1, s2, s3]

    def start(row, q):
        return pltpu.async_copy(
            x_ref.at[pl.ds(row, 8), pl.ds(q * _QW, _QW)],
            bufs[q],
            sems[q],
        )

    def sweep(blk_row_base, q, buf, carry):
        rs0, xt0, x00 = carry
        t_bs = [
            tb_v[pl.ds((blk_row_base + s) * _L, _L)] for s in range(8)
        ]
        m_bs = [jnp.where(t_bs[s] >= 0, 1.0, 0.0) for s in range(8)]
        if q == 0:
            for s in range(8):
                fc = buf[s, pl.ds(0, _L)]
                x00 = x00 + jnp.where(
                    lanes == 0, m_bs[s] * (_C + _EPS * fc), 0.0
                )

        def body(c2, c, _buf=buf, _t=t_bs, _m=m_bs, _q=q):
            rs, xt = c
            voff = _q * _QW + c2 * 128
            v_cs = [lanes_c[cc] + voff for cc in range(8)]
            for s in range(8):
                for cc in range(8):
                    chunk = _buf[s, pl.ds(c2 * 128 + cc * _L, _L)]
                    rs = rs + chunk * _m[s]
                    xt = xt + jnp.where(v_cs[cc] == _t[s], chunk, 0.0)
            return rs, xt

        rs0, xt0 = lax.fori_loop(0, _QW // 128, body, (rs0, xt0))
        return rs0, xt0, x00

    for q in range(4):
        start(r0, q)

    def outer(i, carry):
        for q in range(4):
            pltpu.make_async_copy(
                x_ref.at[pl.ds(r0, 8), pl.ds(q * _QW, _QW)],
                bufs[q],
                sems[q],
            ).wait()
            carry = sweep(i * 8, q, bufs[q], carry)
            start(r0 + (i + 1) * 8, q)
        return carry

    zero = jnp.zeros((_L,), jnp.float32)
    carry = lax.fori_loop(0, nblocks - 1, outer, (zero, zero, zero))

    for q in range(4):
        pltpu.make_async_copy(
            x_ref.at[pl.ds(r0, 8), pl.ds(q * _QW, _QW)],
            bufs[q],
            sems[q],
        ).wait()
        carry = sweep((nblocks - 1) * 8, q, bufs[q], carry)

    rs_acc, xt_acc, x0c_acc = carry
    res_v[...] = -_EPS * rs_acc - (_CONF - _EPS) * xt_acc + x0c_acc
    pltpu.sync_copy(res_v, out_ref.at[wid])


def _sc_tail(x2, t_bcast):
    body = functools.partial(
        pl.kernel,
        out_type=jax.ShapeDtypeStruct((_NW, _L), jnp.float32),
        mesh=plsc.VectorSubcoreMesh(core_axis_name="c", subcore_axis_name="s"),
        scratch_types=[
            pltpu.VMEM((_W_ROWS * _L,), jnp.int32),
            pltpu.VMEM((8, _QW), jnp.float32),
            pltpu.VMEM((8, _QW), jnp.float32),
            pltpu.VMEM((8, _QW), jnp.float32),
            pltpu.VMEM((8, _QW), jnp.float32),
            pltpu.VMEM((_L,), jnp.float32),
            pltpu.SemaphoreType.DMA,
            pltpu.SemaphoreType.DMA,
            pltpu.SemaphoreType.DMA,
            pltpu.SemaphoreType.DMA,
        ],
    )(_sc_body)
    return body(x2, t_bcast)


def kernel(x, target):
    B, S, V = x.shape
    rows = B * S
    rb = _ROWS_PER_BLOCK
    tc_rows = rows - _SC_ROWS
    nblk = tc_rows // rb
    x2 = x.reshape(rows, V)
    t_flat = target.reshape(rows).astype(jnp.int32)
    t_eff = jnp.where(t_flat == _PAD, -1, t_flat)
    tb_all = jnp.broadcast_to(t_eff[:, None], (rows, 128))

    t_bcast = jnp.broadcast_to(
        t_eff[tc_rows:, None], (_SC_ROWS, _L)
    ).reshape(_SC_ROWS * _L)
    sc_partials = _sc_tail(x2, t_bcast)

    dense = pl.pallas_call(
        _tc_dense_block,
        grid=(nblk,),
        in_specs=[
            pl.BlockSpec((rb, V), lambda i: (i, 0)),
            pl.BlockSpec((rb, 128), lambda i: (i, 0)),
        ],
        out_specs=pl.BlockSpec(
            (1, 1), lambda i: (0, 0), memory_space=pltpu.SMEM
        ),
        out_shape=jax.ShapeDtypeStruct((1, 1), jnp.float32),
    )(x2, tb_all)

    return dense[0, 0] + jnp.sum(sc_partials)


# speedup vs baseline: 1.0170x; 1.0170x over previous
import functools
import math

import jax
import jax.numpy as jnp
from jax import lax
from jax.experimental import pallas as pl
from jax.experimental.pallas import tpu as pltpu
from jax.experimental.pallas import tpu_sc as plsc

_SIZE = 8192
_PAD = 0
_SMOOTHING = 0.1
_CONF = 1.0 - _SMOOTHING
_EPS = _SMOOTHING / (_SIZE - 2)
_C = (_SIZE - 2) * _EPS * math.log(_EPS) + _CONF * math.log(_CONF)

_ROWS_PER_BLOCK = 256

_NC, _NS, _L = 2, 16, 16
_NW = _NC * _NS
_SC_ROWS = 1024
_W_ROWS = _SC_ROWS // _NW


def _tc_dense_block(x_ref, t_ref, out_ref):
    i = pl.program_id(0)

    @pl.when(i == 0)
    def _():
        out_ref[0, 0] = 0.0

    xb = x_ref[...]
    t = t_ref[:, 0:1]
    rowsum = jnp.sum(xb, axis=1, keepdims=True)
    x0 = xb[:, 0:1]
    rb, v = xb.shape
    vocab_ids = lax.broadcasted_iota(jnp.int32, (rb, v), 1)
    xt = jnp.sum(jnp.where(vocab_ids == t, xb, 0.0), axis=1, keepdims=True)
    contrib = jnp.where(
        t >= 0,
        _C - _EPS * rowsum + _EPS * x0 - (_CONF - _EPS) * xt,
        0.0,
    )
    out_ref[0, 0] += jnp.sum(contrib)


_QW = _SIZE // 4
_NBUF = 4


def _sc_body(x_ref, te_ref, out_ref, tb_v, b0, b1, b2, b3, res_v, s0, s1, s2, s3):
    wid = lax.axis_index("s") * _NC + lax.axis_index("c")
    rows = x_ref.shape[0]
    r0 = rows - _SC_ROWS + wid * _W_ROWS
    nblocks = _W_ROWS // 8
    pltpu.sync_copy(
        te_ref.at[pl.ds(wid * _W_ROWS * _L, _W_ROWS * _L)], tb_v
    )
    lanes = lax.iota(jnp.int32, _L)
    lanes_c = [lanes + c * _L for c in range(8)]
    bufs = [b0, b1, b2, b3]
    sems = [s0, s1, s2, s3]

    def start(row, q):
        return pltpu.async_copy(
            x_ref.at[pl.ds(row, 8), pl.ds(q * _QW, _QW)],
            bufs[q],
            sems[q],
        )

    def sweep(blk_row_base, q, buf, carry):
        rs0, xt0, x00 = carry
        t_bs = [
            tb_v[pl.ds((blk_row_base + s) * _L, _L)] for s in range(8)
        ]
        m_bs = [jnp.where(t_bs[s] >= 0, 1.0, 0.0) for s in range(8)]
        if q == 0:
            for s in range(8):
                fc = buf[s, pl.ds(0, _L)]
                x00 = x00 + jnp.where(
                    lanes == 0, m_bs[s] * (_C + _EPS * fc), 0.0
                )

        def body(c2, c, _buf=buf, _t=t_bs, _m=m_bs, _q=q):
            rs, xt = c
            voff = _q * _QW + c2 * 128
            v_cs = [lanes_c[cc] + voff for cc in range(8)]
            for s in range(8):
                for cc in range(8):
                    chunk = _buf[s, pl.ds(c2 * 128 + cc * _L, _L)]
                    rs = rs + chunk * _m[s]
                    xt = xt + jnp.where(v_cs[cc] == _t[s], chunk, 0.0)
            return rs, xt

        rs0, xt0 = lax.fori_loop(0, _QW // 128, body, (rs0, xt0))
        return rs0, xt0, x00

    for q in range(4):
        start(r0, q)

    def outer(i, carry):
        for q in range(4):
            pltpu.make_async_copy(
                x_ref.at[pl.ds(r0, 8), pl.ds(q * _QW, _QW)],
                bufs[q],
                sems[q],
            ).wait()
            carry = sweep(i * 8, q, bufs[q], carry)
            start(r0 + (i + 1) * 8, q)
        return carry

    zero = jnp.zeros((_L,), jnp.float32)
    carry = lax.fori_loop(0, nblocks - 1, outer, (zero, zero, zero))

    for q in range(4):
        pltpu.make_async_copy(
            x_ref.at[pl.ds(r0, 8), pl.ds(q * _QW, _QW)],
            bufs[q],
            sems[q],
        ).wait()
        carry = sweep((nblocks - 1) * 8, q, bufs[q], carry)

    rs_acc, xt_acc, x0c_acc = carry
    res_v[...] = -_EPS * rs_acc - (_CONF - _EPS) * xt_acc + x0c_acc
    pltpu.sync_copy(res_v, out_ref.at[wid])


def _sc_tail(x2, t_bcast):
    body = functools.partial(
        pl.kernel,
        out_type=jax.ShapeDtypeStruct((_NW, _L), jnp.float32),
        mesh=plsc.VectorSubcoreMesh(core_axis_name="c", subcore_axis_name="s"),
        scratch_types=[
            pltpu.VMEM((_W_ROWS * _L,), jnp.int32),
            pltpu.VMEM((8, _QW), jnp.float32),
            pltpu.VMEM((8, _QW), jnp.float32),
            pltpu.VMEM((8, _QW), jnp.float32),
            pltpu.VMEM((8, _QW), jnp.float32),
            pltpu.VMEM((_L,), jnp.float32),
            pltpu.SemaphoreType.DMA,
            pltpu.SemaphoreType.DMA,
            pltpu.SemaphoreType.DMA,
            pltpu.SemaphoreType.DMA,
        ],
    )(_sc_body)
    return body(x2, t_bcast)


def kernel(x, target):
    B, S, V = x.shape
    rows = B * S
    rb = _ROWS_PER_BLOCK
    tc_rows = rows - _SC_ROWS
    nblk = tc_rows // rb
    x2 = x.reshape(rows, V)
    t_flat = target.reshape(rows).astype(jnp.int32)
    t_eff = jnp.where(t_flat == _PAD, -1, t_flat)
    tb_all = jnp.broadcast_to(t_eff[:, None], (rows, 128))

    t_bcast = jnp.broadcast_to(
        t_eff[tc_rows:, None], (_SC_ROWS, _L)
    ).reshape(_SC_ROWS * _L)
    sc_partials = _sc_tail(x2, t_bcast)

    dense = pl.pallas_call(
        _tc_dense_block,
        grid=(nblk,),
        in_specs=[
            pl.BlockSpec((rb, V), lambda i: (i, 0)),
            pl.BlockSpec((rb, 128), lambda i: (i, 0)),
        ],
        out_specs=pl.BlockSpec(
            (1, 1), lambda i: (0, 0), memory_space=pltpu.SMEM
        ),
        out_shape=jax.ShapeDtypeStruct((1, 1), jnp.float32),
    )(x2, tb_all)

    return dense[0, 0] + jnp.sum(sc_partials)
